# Initial kernel scaffold; baseline (speedup 1.0000x reference)
import jax
import jax.numpy as jnp
from jax.experimental import pallas as pl


def kernel(x, W1, g1, b1, W2, g2, b2):
    raise NotImplementedError("write your pallas kernel here")


# speedup vs baseline: 10.1858x; 10.1858x over previous
import functools

import jax
import jax.numpy as jnp
from jax import lax
from jax.experimental import pallas as pl
from jax.experimental.pallas import tpu as pltpu
from jax.experimental.pallas import tpu_sc as plsc

KNN = 20
EPSV = 1e-5
SLOPE = 0.2
NEGF = -3.0e38

B = 8
N = 2048
CIN = 3
COUT = 64
DPAD = 16
RT = 256
RN = 256
RF = 512
MBLK = KNN * RN
MTOT = B * N * KNN

NW = 32
RPW = MTOT // NW
CHW = 128
CH = RPW // CHW
NBUF = 8
GRP = CH // NBUF


def _topk_body(x_ref, xt_ref, idx_ref):
    b = pl.program_id(0)
    j = pl.program_id(1)
    xb = x_ref[0]
    xx = jnp.sum(xb * xb, axis=0)
    xr = xt_ref[0, pl.ds(j * RT, RT), :]
    inner = lax.dot_general(xr, xb, (((1,), (0,)), ((), ())),
                            preferred_element_type=jnp.float32)
    xxr = jnp.sum(xr * xr, axis=1, keepdims=True)
    sc = -(xxr - 2.0 * inner + xx[None, :])
    iota = lax.broadcasted_iota(jnp.int32, (RT, N), 1)
    cols = []
    for _ in range(KNN):
        m = jnp.max(sc, axis=1, keepdims=True)
        cand = jnp.where(sc >= m, iota, N)
        col = jnp.min(cand, axis=1, keepdims=True)
        cols.append(col)
        sc = jnp.where(iota == col, NEGF, sc)
    idx_ref[0] = jnp.concatenate(cols, axis=1) + b * N


def _topk_call(x, xt):
    return pl.pallas_call(
        _topk_body,
        grid=(B, N // RT),
        in_specs=[
            pl.BlockSpec((1, CIN, N), lambda b, j: (b, 0, 0)),
            pl.BlockSpec((1, N, CIN), lambda b, j: (b, 0, 0)),
        ],
        out_specs=pl.BlockSpec((1, RT, KNN), lambda b, j: (b, j, 0)),
        out_shape=jax.ShapeDtypeStruct((B, N, KNN), jnp.int32),
    )(x, xt)


def _sc_gather_body(table_hbm, idx_hbm, out_hbm, idx_v, bufs, gsem, ssem):
    wid = lax.axis_index("s") * 2 + lax.axis_index("c")
    pltpu.sync_copy(idx_hbm.at[wid], idx_v)
    row0 = wid * RPW

    def group(g, carry):
        gets = []
        for t in range(NBUF):
            j = g * NBUF + t
            gets.append(pltpu.async_copy(
                table_hbm.at[idx_v.at[j]], bufs.at[t], gsem))
        for cp in gets:
            cp.wait()
        puts = []
        for t in range(NBUF):
            j = g * NBUF + t
            puts.append(pltpu.async_copy(
                bufs.at[t], out_hbm.at[pl.ds(row0 + j * CHW, CHW)], ssem))
        for cp in puts:
            cp.wait()
        return carry

    lax.fori_loop(0, GRP, group, 0)


def _sc_gather(table, idx3):
    f = pl.kernel(
        _sc_gather_body,
        out_type=jax.ShapeDtypeStruct((MTOT, DPAD), jnp.float32),
        mesh=plsc.VectorSubcoreMesh(core_axis_name="c", subcore_axis_name="s"),
        scratch_types=[
            pltpu.VMEM((CH, CHW), jnp.int32),
            pltpu.VMEM((NBUF, CHW, DPAD), jnp.float32),
            pltpu.SemaphoreType.DMA,
            pltpu.SemaphoreType.DMA,
        ],
        compiler_params=pltpu.CompilerParams(use_tc_tiling_on_sc=False),
    )
    return f(table, idx3)


def _stats1_body(g_ref, xt_ref, w1p_ref, w1d_ref, s_ref, q_ref):
    b = pl.program_id(0)
    j = pl.program_id(1)
    g2d = g_ref[:, 0].reshape(MBLK, DPAD)
    y1a = lax.dot_general(g2d, w1p_ref[...], (((1,), (0,)), ((), ())),
                          preferred_element_type=jnp.float32)
    yi = lax.dot_general(xt_ref[0], w1d_ref[...], (((1,), (0,)), ((), ())),
                         preferred_element_type=jnp.float32)
    y1 = y1a.reshape(KNN, RN, COUT) + yi[None, :, :]
    ps = jnp.sum(y1, axis=(0, 1))
    pq = jnp.sum(y1 * y1, axis=(0, 1))

    @pl.when(j == 0)
    def _init():
        s_ref[pl.ds(b, 1), :] = ps[None, :]
        q_ref[pl.ds(b, 1), :] = pq[None, :]

    @pl.when(j > 0)
    def _acc():
        s_ref[pl.ds(b, 1), :] += ps[None, :]
        q_ref[pl.ds(b, 1), :] += pq[None, :]


def _stats1_call(garr, xt, w1p, w1d):
    return pl.pallas_call(
        _stats1_body,
        grid=(B, N // RN),
        in_specs=[
            pl.BlockSpec((KNN, 1, RN, DPAD), lambda b, j: (0, b, j, 0)),
            pl.BlockSpec((1, RN, CIN), lambda b, j: (b, j, 0)),
            pl.BlockSpec((DPAD, COUT), lambda b, j: (0, 0)),
            pl.BlockSpec((CIN, COUT), lambda b, j: (0, 0)),
        ],
        out_specs=[
            pl.BlockSpec((B, COUT), lambda b, j: (0, 0)),
            pl.BlockSpec((B, COUT), lambda b, j: (0, 0)),
        ],
        out_shape=[
            jax.ShapeDtypeStruct((B, COUT), jnp.float32),
            jax.ShapeDtypeStruct((B, COUT), jnp.float32),
        ],
    )(garr, xt, w1p, w1d)


def _stats2_body(g_ref, xt_ref, w1p_ref, w1d_ref, gb1_ref, s1_ref, q1_ref,
                 w2_ref, s_ref, q_ref, mx_ref, mn_ref):
    b = pl.program_id(0)
    j = pl.program_id(1)
    mean1 = jnp.sum(s1_ref[...], axis=0) * (1.0 / MTOT)
    var1 = jnp.sum(q1_ref[...], axis=0) * (1.0 / MTOT) - mean1 * mean1
    a1 = gb1_ref[0] * lax.rsqrt(var1 + EPSV)
    c1 = gb1_ref[1] - mean1 * a1

    g2d = g_ref[:, 0].reshape(MBLK, DPAD)
    y1a = lax.dot_general(g2d, w1p_ref[...], (((1,), (0,)), ((), ())),
                          preferred_element_type=jnp.float32)
    yi = lax.dot_general(xt_ref[0], w1d_ref[...], (((1,), (0,)), ((), ())),
                         preferred_element_type=jnp.float32)
    y1 = y1a.reshape(KNN, RN, COUT) + yi[None, :, :]
    z1 = a1[None, None, :] * y1 + c1[None, None, :]
    z1 = jnp.where(z1 > 0, z1, SLOPE * z1)
    y2 = lax.dot_general(z1.reshape(MBLK, COUT), w2_ref[...],
                         (((1,), (0,)), ((), ())),
                         preferred_element_type=jnp.float32)
    ps = jnp.sum(y2, axis=0)
    pq = jnp.sum(y2 * y2, axis=0)
    y3 = y2.reshape(KNN, RN, COUT)
    mx_ref[0] = jnp.max(y3, axis=0)
    mn_ref[0] = jnp.min(y3, axis=0)

    @pl.when(j == 0)
    def _init():
        s_ref[pl.ds(b, 1), :] = ps[None, :]
        q_ref[pl.ds(b, 1), :] = pq[None, :]

    @pl.when(j > 0)
    def _acc():
        s_ref[pl.ds(b, 1), :] += ps[None, :]
        q_ref[pl.ds(b, 1), :] += pq[None, :]


def _stats2_call(garr, xt, w1p, w1d, gb1, s1, q1, w2):
    return pl.pallas_call(
        _stats2_body,
        grid=(B, N // RN),
        in_specs=[
            pl.BlockSpec((KNN, 1, RN, DPAD), lambda b, j: (0, b, j, 0)),
            pl.BlockSpec((1, RN, CIN), lambda b, j: (b, j, 0)),
            pl.BlockSpec((DPAD, COUT), lambda b, j: (0, 0)),
            pl.BlockSpec((CIN, COUT), lambda b, j: (0, 0)),
            pl.BlockSpec((2, COUT), lambda b, j: (0, 0)),
            pl.BlockSpec((B, COUT), lambda b, j: (0, 0)),
            pl.BlockSpec((B, COUT), lambda b, j: (0, 0)),
            pl.BlockSpec((COUT, COUT), lambda b, j: (0, 0)),
        ],
        out_specs=[
            pl.BlockSpec((B, COUT), lambda b, j: (0, 0)),
            pl.BlockSpec((B, COUT), lambda b, j: (0, 0)),
            pl.BlockSpec((1, RN, COUT), lambda b, j: (b, j, 0)),
            pl.BlockSpec((1, RN, COUT), lambda b, j: (b, j, 0)),
        ],
        out_shape=[
            jax.ShapeDtypeStruct((B, COUT), jnp.float32),
            jax.ShapeDtypeStruct((B, COUT), jnp.float32),
            jax.ShapeDtypeStruct((B, N, COUT), jnp.float32),
            jax.ShapeDtypeStruct((B, N, COUT), jnp.float32),
        ],
    )(garr, xt, w1p, w1d, gb1, s1, q1, w2)


def _final_body(mx_ref, mn_ref, gb2_ref, s2_ref, q2_ref, o_ref):
    mean2 = jnp.sum(s2_ref[...], axis=0) * (1.0 / MTOT)
    var2 = jnp.sum(q2_ref[...], axis=0) * (1.0 / MTOT) - mean2 * mean2
    a2 = gb2_ref[0] * lax.rsqrt(var2 + EPSV)
    c2 = gb2_ref[1] - mean2 * a2
    mx = mx_ref[0]
    mn = mn_ref[0]
    t = jnp.maximum(a2[None, :] * mx, a2[None, :] * mn) + c2[None, :]
    t = jnp.where(t > 0, t, SLOPE * t)
    o_ref[0] = t.T


def _final_call(mx, mn, gb2, s2, q2):
    return pl.pallas_call(
        _final_body,
        grid=(B, N // RF),
        in_specs=[
            pl.BlockSpec((1, RF, COUT), lambda b, j: (b, j, 0)),
            pl.BlockSpec((1, RF, COUT), lambda b, j: (b, j, 0)),
            pl.BlockSpec((2, COUT), lambda b, j: (0, 0)),
            pl.BlockSpec((B, COUT), lambda b, j: (0, 0)),
            pl.BlockSpec((B, COUT), lambda b, j: (0, 0)),
        ],
        out_specs=pl.BlockSpec((1, COUT, RF), lambda b, j: (b, 0, j)),
        out_shape=jax.ShapeDtypeStruct((B, COUT, N), jnp.float32),
    )(mx, mn, gb2, s2, q2)


def kernel(x, W1, g1, b1, W2, g2, b2):
    x = x.astype(jnp.float32)
    xt = jnp.transpose(x, (0, 2, 1))
    idx = _topk_call(x, xt)
    idx3 = jnp.transpose(idx, (2, 0, 1)).reshape(NW, CH, CHW)
    table = jnp.pad(xt.reshape(B * N, CIN), ((0, 0), (0, DPAD - CIN)))
    garr = _sc_gather(table, idx3).reshape(KNN, B, N, DPAD)
    w1p = jnp.pad(W1[:, :CIN], ((0, 0), (0, DPAD - CIN))).T
    w1d = (W1[:, CIN:] - W1[:, :CIN]).T
    s1, q1 = _stats1_call(garr, xt, w1p, w1d)
    gb1 = jnp.stack([g1, b1])
    s2, q2, mx, mn = _stats2_call(garr, xt, w1p, w1d, gb1, s1, q1, W2.T)
    gb2 = jnp.stack([g2, b2])
    return _final_call(mx, mn, gb2, s2, q2)


# speedup vs baseline: 13.2119x; 1.2971x over previous
import functools

import jax
import jax.numpy as jnp
from jax import lax
from jax.experimental import pallas as pl
from jax.experimental.pallas import tpu as pltpu
from jax.experimental.pallas import tpu_sc as plsc

KNN = 20
EPSV = 1e-5
SLOPE = 0.2
NEGF = -3.0e38

B = 8
N = 2048
CIN = 3
COUT = 64
DPAD = 16
RT = 256
RN = 256
RF = 512
MBLK = KNN * RN
MTOT = B * N * KNN

NW = 32
RPW = MTOT // NW
CHW = 128
CH = RPW // CHW
NBUF = 8
GRP = CH // NBUF


def _topk_body(x_ref, xt_ref, idx_ref):
    b = pl.program_id(0)
    j = pl.program_id(1)
    xb = x_ref[0]
    xx = jnp.sum(xb * xb, axis=0)
    xr = xt_ref[0, pl.ds(j * RT, RT), :]
    inner = lax.dot_general(xr, xb, (((1,), (0,)), ((), ())),
                            preferred_element_type=jnp.float32)
    xxr = jnp.sum(xr * xr, axis=1, keepdims=True)
    sc = -(xxr - 2.0 * inner + xx[None, :])
    iota = lax.broadcasted_iota(jnp.int32, (RT, N), 1)
    cols = []
    m = jnp.max(sc, axis=1, keepdims=True)
    for t in range(KNN):
        cand = jnp.where(sc == m, iota, N)
        cols.append(jnp.min(cand, axis=1, keepdims=True))
        if t < KNN - 1:
            nxt = jnp.where(sc >= m, NEGF, sc)
            m = jnp.max(nxt, axis=1, keepdims=True)
    idx_ref[0] = jnp.concatenate(cols, axis=1) + b * N


def _topk_call(x, xt):
    return pl.pallas_call(
        _topk_body,
        grid=(B, N // RT),
        in_specs=[
            pl.BlockSpec((1, CIN, N), lambda b, j: (b, 0, 0)),
            pl.BlockSpec((1, N, CIN), lambda b, j: (b, 0, 0)),
        ],
        out_specs=pl.BlockSpec((1, RT, KNN), lambda b, j: (b, j, 0)),
        out_shape=jax.ShapeDtypeStruct((B, N, KNN), jnp.int32),
    )(x, xt)


def _sc_gather_body(table_hbm, idx_hbm, out_hbm, idx_v, bufs, gsem, ssem):
    wid = lax.axis_index("s") * 2 + lax.axis_index("c")
    pltpu.sync_copy(idx_hbm.at[wid], idx_v)
    row0 = wid * RPW

    def group(g, carry):
        gets = []
        for t in range(NBUF):
            j = g * NBUF + t
            gets.append(pltpu.async_copy(
                table_hbm.at[idx_v.at[j]], bufs.at[t], gsem))
        for cp in gets:
            cp.wait()
        puts = []
        for t in range(NBUF):
            j = g * NBUF + t
            puts.append(pltpu.async_copy(
                bufs.at[t], out_hbm.at[pl.ds(row0 + j * CHW, CHW)], ssem))
        for cp in puts:
            cp.wait()
        return carry

    lax.fori_loop(0, GRP, group, 0)


def _sc_gather(table, idx3):
    f = pl.kernel(
        _sc_gather_body,
        out_type=jax.ShapeDtypeStruct((MTOT, DPAD), jnp.float32),
        mesh=plsc.VectorSubcoreMesh(core_axis_name="c", subcore_axis_name="s"),
        scratch_types=[
            pltpu.VMEM((CH, CHW), jnp.int32),
            pltpu.VMEM((NBUF, CHW, DPAD), jnp.float32),
            pltpu.SemaphoreType.DMA,
            pltpu.SemaphoreType.DMA,
        ],
        compiler_params=pltpu.CompilerParams(use_tc_tiling_on_sc=False),
    )
    return f(table, idx3)


RN8 = RN // 8
PLN = 8 * COUT
PBLK = KNN * RN8


def _fold8(v):
    r = v[0:COUT]
    for s in range(1, 8):
        r = r + v[s * COUT:(s + 1) * COUT]
    return r


def _stats1_body(g_ref, xt8_ref, w1p8_ref, w1d8_ref, s_ref, q_ref):
    b = pl.program_id(0)
    j = pl.program_id(1)
    g2 = g_ref[:, 0].reshape(PBLK, 8 * DPAD)
    y1a = lax.dot_general(g2, w1p8_ref[...], (((1,), (0,)), ((), ())),
                          preferred_element_type=jnp.float32)
    yi = lax.dot_general(xt8_ref[0], w1d8_ref[...], (((1,), (0,)), ((), ())),
                         preferred_element_type=jnp.float32)
    y1 = y1a.reshape(KNN, RN8, PLN) + yi[None, :, :]
    y1 = y1.reshape(PBLK, PLN)
    ps = _fold8(jnp.sum(y1, axis=0))
    pq = _fold8(jnp.sum(y1 * y1, axis=0))

    @pl.when(j == 0)
    def _init():
        s_ref[pl.ds(b, 1), :] = ps[None, :]
        q_ref[pl.ds(b, 1), :] = pq[None, :]

    @pl.when(j > 0)
    def _acc():
        s_ref[pl.ds(b, 1), :] += ps[None, :]
        q_ref[pl.ds(b, 1), :] += pq[None, :]


def _stats1_call(garr, xt8, w1p8, w1d8):
    return pl.pallas_call(
        _stats1_body,
        grid=(B, N // RN),
        in_specs=[
            pl.BlockSpec((KNN, 1, RN8, 8 * DPAD), lambda b, j: (0, b, j, 0)),
            pl.BlockSpec((1, RN8, 8 * CIN), lambda b, j: (b, j, 0)),
            pl.BlockSpec((8 * DPAD, PLN), lambda b, j: (0, 0)),
            pl.BlockSpec((8 * CIN, PLN), lambda b, j: (0, 0)),
        ],
        out_specs=[
            pl.BlockSpec((B, COUT), lambda b, j: (0, 0)),
            pl.BlockSpec((B, COUT), lambda b, j: (0, 0)),
        ],
        out_shape=[
            jax.ShapeDtypeStruct((B, COUT), jnp.float32),
            jax.ShapeDtypeStruct((B, COUT), jnp.float32),
        ],
    )(garr, xt8, w1p8, w1d8)


def _stats2_body(g_ref, xt8_ref, w1p8_ref, w1d8_ref, gb1_ref, s1_ref, q1_ref,
                 w2b_ref, s_ref, q_ref, mx_ref, mn_ref):
    b = pl.program_id(0)
    j = pl.program_id(1)
    mean1 = jnp.sum(s1_ref[...], axis=0) * (1.0 / MTOT)
    var1 = jnp.sum(q1_ref[...], axis=0) * (1.0 / MTOT) - mean1 * mean1
    a1 = gb1_ref[0] * lax.rsqrt(var1 + EPSV)
    c1 = gb1_ref[1] - mean1 * a1
    a1t = jnp.concatenate([a1] * 8)
    c1t = jnp.concatenate([c1] * 8)

    g2 = g_ref[:, 0].reshape(PBLK, 8 * DPAD)
    y1a = lax.dot_general(g2, w1p8_ref[...], (((1,), (0,)), ((), ())),
                          preferred_element_type=jnp.float32)
    yi = lax.dot_general(xt8_ref[0], w1d8_ref[...], (((1,), (0,)), ((), ())),
                         preferred_element_type=jnp.float32)
    y1 = y1a.reshape(KNN, RN8, PLN) + yi[None, :, :]
    z1 = a1t[None, None, :] * y1 + c1t[None, None, :]
    z1 = jnp.where(z1 > 0, z1, SLOPE * z1)
    y2 = lax.dot_general(z1.reshape(PBLK, PLN), w2b_ref[...],
                         (((1,), (0,)), ((), ())),
                         preferred_element_type=jnp.float32)
    ps = _fold8(jnp.sum(y2, axis=0))
    pq = _fold8(jnp.sum(y2 * y2, axis=0))
    y3 = y2.reshape(KNN, RN8, PLN)
    mx8 = jnp.max(y3, axis=0)
    mn8 = jnp.min(y3, axis=0)
    for s in range(8):
        mx_ref[0, s] = mx8[:, s * COUT:(s + 1) * COUT]
        mn_ref[0, s] = mn8[:, s * COUT:(s + 1) * COUT]

    @pl.when(j == 0)
    def _init():
        s_ref[pl.ds(b, 1), :] = ps[None, :]
        q_ref[pl.ds(b, 1), :] = pq[None, :]

    @pl.when(j > 0)
    def _acc():
        s_ref[pl.ds(b, 1), :] += ps[None, :]
        q_ref[pl.ds(b, 1), :] += pq[None, :]


def _stats2_call(garr, xt8, w1p8, w1d8, gb1, s1, q1, w2b):
    return pl.pallas_call(
        _stats2_body,
        grid=(B, N // RN),
        in_specs=[
            pl.BlockSpec((KNN, 1, RN8, 8 * DPAD), lambda b, j: (0, b, j, 0)),
            pl.BlockSpec((1, RN8, 8 * CIN), lambda b, j: (b, j, 0)),
            pl.BlockSpec((8 * DPAD, PLN), lambda b, j: (0, 0)),
            pl.BlockSpec((8 * CIN, PLN), lambda b, j: (0, 0)),
            pl.BlockSpec((2, COUT), lambda b, j: (0, 0)),
            pl.BlockSpec((B, COUT), lambda b, j: (0, 0)),
            pl.BlockSpec((B, COUT), lambda b, j: (0, 0)),
            pl.BlockSpec((PLN, PLN), lambda b, j: (0, 0)),
        ],
        out_specs=[
            pl.BlockSpec((B, COUT), lambda b, j: (0, 0)),
            pl.BlockSpec((B, COUT), lambda b, j: (0, 0)),
            pl.BlockSpec((1, 8, RN8, COUT), lambda b, j: (b, 0, j, 0)),
            pl.BlockSpec((1, 8, RN8, COUT), lambda b, j: (b, 0, j, 0)),
        ],
        out_shape=[
            jax.ShapeDtypeStruct((B, COUT), jnp.float32),
            jax.ShapeDtypeStruct((B, COUT), jnp.float32),
            jax.ShapeDtypeStruct((B, 8, N // 8, COUT), jnp.float32),
            jax.ShapeDtypeStruct((B, 8, N // 8, COUT), jnp.float32),
        ],
    )(garr, xt8, w1p8, w1d8, gb1, s1, q1, w2b)


N8 = N // 8


def _final_body(mx_ref, mn_ref, gb2_ref, s2_ref, q2_ref, o_ref):
    mean2 = jnp.sum(s2_ref[...], axis=0) * (1.0 / MTOT)
    var2 = jnp.sum(q2_ref[...], axis=0) * (1.0 / MTOT) - mean2 * mean2
    a2 = gb2_ref[0] * lax.rsqrt(var2 + EPSV)
    c2 = gb2_ref[1] - mean2 * a2
    mx = mx_ref[0]
    mn = mn_ref[0]
    a2b = a2[None, None, :]
    t = jnp.maximum(a2b * mx, a2b * mn) + c2[None, None, :]
    t = jnp.where(t > 0, t, SLOPE * t)
    for s in range(8):
        o_ref[0, :, pl.ds(s * N8, N8)] = t[s].T


def _final_call(mx, mn, gb2, s2, q2):
    return pl.pallas_call(
        _final_body,
        grid=(B,),
        in_specs=[
            pl.BlockSpec((1, 8, N8, COUT), lambda b: (b, 0, 0, 0)),
            pl.BlockSpec((1, 8, N8, COUT), lambda b: (b, 0, 0, 0)),
            pl.BlockSpec((2, COUT), lambda b: (0, 0)),
            pl.BlockSpec((B, COUT), lambda b: (0, 0)),
            pl.BlockSpec((B, COUT), lambda b: (0, 0)),
        ],
        out_specs=pl.BlockSpec((1, COUT, N), lambda b: (b, 0, 0)),
        out_shape=jax.ShapeDtypeStruct((B, COUT, N), jnp.float32),
    )(mx, mn, gb2, s2, q2)


def kernel(x, W1, g1, b1, W2, g2, b2):
    x = x.astype(jnp.float32)
    xt = jnp.transpose(x, (0, 2, 1))
    idx = _topk_call(x, xt)
    idxP = jnp.transpose(idx, (2, 0, 1)).reshape(KNN, B, 8, N // 8)
    idx3 = jnp.transpose(idxP, (0, 1, 3, 2)).reshape(NW, CH, CHW)
    table = jnp.pad(xt.reshape(B * N, CIN), ((0, 0), (0, DPAD - CIN)))
    garr = _sc_gather(table, idx3).reshape(KNN, B, N // 8, 8 * DPAD)
    xt8 = jnp.transpose(xt.reshape(B, 8, N // 8, CIN),
                        (0, 2, 1, 3)).reshape(B, N // 8, 8 * CIN)
    eye8 = jnp.eye(8, dtype=jnp.float32)
    w1p = jnp.pad(W1[:, :CIN], ((0, 0), (0, DPAD - CIN))).T
    w1d = (W1[:, CIN:] - W1[:, :CIN]).T
    w1p8 = jnp.kron(eye8, w1p)
    w1d8 = jnp.kron(eye8, w1d)
    w2b = jnp.kron(eye8, W2.T)
    s1, q1 = _stats1_call(garr, xt8, w1p8, w1d8)
    gb1 = jnp.stack([g1, b1])
    s2, q2, mx, mn = _stats2_call(garr, xt8, w1p8, w1d8, gb1, s1, q1, w2b)
    gb2 = jnp.stack([g2, b2])
    return _final_call(mx, mn, gb2, s2, q2)
